# Initial kernel scaffold; baseline (speedup 1.0000x reference)
import functools

import jax
import jax.numpy as jnp
from jax import lax
from jax.experimental import pallas as pl
from jax.experimental.pallas import tpu as pltpu
from jax.experimental.pallas import tpu_sc as plsc

N = 10000
E = 320000
D = 128
G = 64
C = 10
H = 768

NC = 2
NS = 16
NW = NC * NS
ET = E // NW
CHUNK = 80
NCHUNK = ET // CHUNK
SLAB = N // NS

BLK = 1000
NBLK = N // BLK


def _segsum_body(x_hbm, src_hbm, dst_hbm, out_hbm,
                 src_t, dst_t, rows0, rows1, acc, g0, g1, s0, s1):
    c = lax.axis_index("c")
    s = lax.axis_index("s")
    w = c * NS + s
    row0 = s * SLAB
    pltpu.sync_copy(x_hbm.at[pl.ds(row0, SLAB)], acc.at[pl.ds(row0, SLAB)])
    pltpu.sync_copy(src_hbm.at[w], src_t)
    pltpu.sync_copy(dst_hbm.at[w], dst_t)
    plsc.subcore_barrier()

    def pair(i, carry):
        j0 = 2 * i
        j1 = j0 + 1
        cp0 = pltpu.async_copy(x_hbm.at[src_t.at[j0]], rows0, g0)
        cp1 = pltpu.async_copy(x_hbm.at[src_t.at[j1]], rows1, g1)
        cp0.wait()
        sc0 = pltpu.async_copy(rows0, acc.at[dst_t.at[j0]], s0, add=True)
        cp1.wait()
        sc1 = pltpu.async_copy(rows1, acc.at[dst_t.at[j1]], s1, add=True)
        sc0.wait()
        sc1.wait()
        return carry

    lax.fori_loop(0, NCHUNK // 2, pair, 0)
    jt = NCHUNK - 1
    pltpu.async_copy(x_hbm.at[src_t.at[jt]], rows0, g0).wait()
    pltpu.sync_copy(rows0, acc.at[dst_t.at[jt]], add=True)
    plsc.subcore_barrier()
    pltpu.sync_copy(acc.at[pl.ds(row0, SLAB)], out_hbm.at[c, pl.ds(row0, SLAB)])


_segsum = pl.kernel(
    _segsum_body,
    out_type=jax.ShapeDtypeStruct((NC, N, D), jnp.float32),
    mesh=plsc.VectorSubcoreMesh(core_axis_name="c", subcore_axis_name="s"),
    scratch_types=[
        pltpu.VMEM((NCHUNK, CHUNK), jnp.int32),
        pltpu.VMEM((NCHUNK, CHUNK), jnp.int32),
        pltpu.VMEM((CHUNK, D), jnp.float32),
        pltpu.VMEM((CHUNK, D), jnp.float32),
        pltpu.VMEM_SHARED((N, D), jnp.float32),
        pltpu.SemaphoreType.DMA,
        pltpu.SemaphoreType.DMA,
        pltpu.SemaphoreType.DMA,
        pltpu.SemaphoreType.DMA,
    ],
    name="segsum_sc",
)


def _tc1_body(x_ref, p0_ref, p1_ref, w1a_ref, b1a_ref, w1b_ref, b1b_ref,
              lng_ref, lnb_ref, o_ref):
    h0 = p0_ref[...] + p1_ref[...] - x_ref[...]
    t = jnp.dot(h0, w1a_ref[...], preferred_element_type=jnp.float32)
    t = jnp.maximum(t + b1a_ref[...], 0.0)
    h1 = jnp.dot(t, w1b_ref[...], preferred_element_type=jnp.float32)
    h1 = h1 + b1b_ref[...]
    m = jnp.sum(h1, axis=-1, keepdims=True) * (1.0 / D)
    d = h1 - m
    v = jnp.sum(d * d, axis=-1, keepdims=True) * (1.0 / D)
    hn = d * lax.rsqrt(v + 1e-5) * lng_ref[...] + lnb_ref[...]
    o_ref[...] = jnp.where(hn > 0, hn, 0.2 * hn)


def _tc1(x, p0, p1, w1a, b1a, w1b, b1b, lng, lnb):
    row_spec = pl.BlockSpec((BLK, D), lambda i: (i, 0))
    full = lambda a: pl.BlockSpec(a.shape, lambda i: (0,) * a.ndim)
    return pl.pallas_call(
        _tc1_body,
        grid=(NBLK,),
        in_specs=[row_spec, row_spec, row_spec,
                  full(w1a), full(b1a), full(w1b), full(b1b),
                  full(lng), full(lnb)],
        out_specs=row_spec,
        out_shape=jax.ShapeDtypeStruct((N, D), jnp.float32),
    )(x, p0, p1, w1a, b1a, w1b, b1b, lng, lnb)


def _tc2_body(h_ref, q0_ref, q1_ref, w2a_ref, b2a_ref, w2b_ref, b2b_ref,
              bt_ref, pooled_ref):
    i = pl.program_id(0)

    @pl.when(i == 0)
    def _init():
        pooled_ref[...] = jnp.full((G, D), -jnp.inf, jnp.float32)

    h0 = q0_ref[...] + q1_ref[...] - h_ref[...]
    t = jnp.dot(h0, w2a_ref[...], preferred_element_type=jnp.float32)
    t = jnp.maximum(t + b2a_ref[...], 0.0)
    h2 = jnp.dot(t, w2b_ref[...], preferred_element_type=jnp.float32)
    h2 = h2 + b2b_ref[...]
    h2 = jnp.where(h2 > 0, h2, 0.2 * h2)
    bt = bt_ref[...]
    rows = [jnp.max(jnp.where(bt == g, h2, -jnp.inf), axis=0)
            for g in range(G)]
    pooled_ref[...] = jnp.maximum(pooled_ref[...], jnp.stack(rows))


def _tc2(h, q0, q1, w2a, b2a, w2b, b2b, batch2d):
    row_spec = pl.BlockSpec((BLK, D), lambda i: (i, 0))
    full = lambda a: pl.BlockSpec(a.shape, lambda i: (0,) * a.ndim)
    return pl.pallas_call(
        _tc2_body,
        grid=(NBLK,),
        in_specs=[row_spec, row_spec, row_spec,
                  full(w2a), full(b2a), full(w2b), full(b2b),
                  pl.BlockSpec((BLK, 1), lambda i: (i, 0))],
        out_specs=pl.BlockSpec((G, D), lambda i: (0, 0)),
        out_shape=jax.ShapeDtypeStruct((G, D), jnp.float32),
    )(h, q0, q1, w2a, b2a, w2b, b2b, batch2d)


def _tc3_body(p_ref, sw1_ref, sb1_ref, slng_ref, slnb_ref, sw2_ref, sb2_ref,
              o_ref):
    z = jnp.dot(p_ref[...], sw1_ref[...], preferred_element_type=jnp.float32)
    z = z + sb1_ref[...]
    m = jnp.sum(z, axis=-1, keepdims=True) * (1.0 / H)
    d = z - m
    v = jnp.sum(d * d, axis=-1, keepdims=True) * (1.0 / H)
    z = d * lax.rsqrt(v + 1e-5) * slng_ref[...] + slnb_ref[...]
    z = jnp.where(z > 0, z, 0.2 * z)
    logits = jnp.dot(z, sw2_ref[...], preferred_element_type=jnp.float32)
    logits = logits + sb2_ref[...]
    mx = jnp.max(logits, axis=-1, keepdims=True)
    e = jnp.exp(logits - mx)
    o_ref[...] = e / jnp.sum(e, axis=-1, keepdims=True)


def _tc3(pooled, sw1, sb1, slng, slnb, sw2, sb2):
    return pl.pallas_call(
        _tc3_body,
        out_shape=jax.ShapeDtypeStruct((G, C), jnp.float32),
    )(pooled, sw1, sb1, slng, slnb, sw2, sb2)


def kernel(x, edge_index, batch, W1a, b1a, W1b, b1b, lng, lnb,
           W2a, b2a, W2b, b2b, Sw1, Sb1, Slng, Slnb, Sw2, Sb2):
    src3 = edge_index[0].reshape(NW, NCHUNK, CHUNK)
    dst3 = edge_index[1].reshape(NW, NCHUNK, CHUNK)
    p = _segsum(x, src3, dst3)
    h = _tc1(x, p[0], p[1], W1a, b1a.reshape(1, D), W1b, b1b.reshape(1, D),
             lng.reshape(1, D), lnb.reshape(1, D))
    q = _segsum(h, src3, dst3)
    pooled = _tc2(h, q[0], q[1], W2a, b2a.reshape(1, D), W2b,
                  b2b.reshape(1, D), batch.reshape(N, 1))
    return _tc3(pooled, Sw1, Sb1.reshape(1, H), Slng.reshape(1, H),
                Slnb.reshape(1, H), Sw2, Sb2.reshape(1, C))


# speedup vs baseline: 6.9436x; 6.9436x over previous
import functools

import jax
import jax.numpy as jnp
from jax import lax
from jax.experimental import pallas as pl
from jax.experimental.pallas import tpu as pltpu
from jax.experimental.pallas import tpu_sc as plsc

N = 10000
E = 320000
D = 128
G = 64
C = 10
H = 768

NC = 2
NS = 16
NW = NC * NS
ET = E // NW
CHUNK = 80
NCHUNK = ET // CHUNK
SLAB = 640
SLAB_TAIL = N - (NS - 1) * SLAB

BLK = 1000
NBLK = N // BLK


def _segsum_body(x_hbm, src_hbm, dst_hbm, out_hbm,
                 src_t, didx0, didx1, rows0, rows1, acc,
                 g0, g1, s0, s1, d0, d1):
    c = lax.axis_index("c")
    s = lax.axis_index("s")
    w = c * NS + s
    row0 = s * SLAB
    pltpu.sync_copy(x_hbm.at[pl.ds(row0, SLAB_TAIL)],
                    acc.at[pl.ds(row0, SLAB_TAIL)])

    @pl.when(s < NS - 1)
    def _init_rest():
        pltpu.sync_copy(x_hbm.at[pl.ds(row0 + SLAB_TAIL, SLAB - SLAB_TAIL)],
                        acc.at[pl.ds(row0 + SLAB_TAIL, SLAB - SLAB_TAIL)])

    pltpu.sync_copy(src_hbm.at[w], src_t)
    plsc.subcore_barrier()

    def pair(i, carry):
        j0 = 2 * i
        j1 = j0 + 1
        dd0 = pltpu.async_copy(dst_hbm.at[pl.ds(w * NCHUNK + j0, 1)], didx0, d0)
        dd1 = pltpu.async_copy(dst_hbm.at[pl.ds(w * NCHUNK + j1, 1)], didx1, d1)
        cp0 = pltpu.async_copy(x_hbm.at[src_t.at[j0]], rows0, g0)
        cp1 = pltpu.async_copy(x_hbm.at[src_t.at[j1]], rows1, g1)
        cp0.wait()
        dd0.wait()
        sc0 = pltpu.async_copy(rows0, acc.at[didx0.at[0]], s0, add=True)
        cp1.wait()
        dd1.wait()
        sc1 = pltpu.async_copy(rows1, acc.at[didx1.at[0]], s1, add=True)
        sc0.wait()
        sc1.wait()
        return carry

    lax.fori_loop(0, NCHUNK // 2, pair, 0)
    jt = NCHUNK - 1
    pltpu.sync_copy(dst_hbm.at[pl.ds(w * NCHUNK + jt, 1)], didx0)
    pltpu.async_copy(x_hbm.at[src_t.at[jt]], rows0, g0).wait()
    pltpu.sync_copy(rows0, acc.at[didx0.at[0]], add=True)
    plsc.subcore_barrier()
    pltpu.sync_copy(acc.at[pl.ds(row0, SLAB_TAIL)],
                    out_hbm.at[c, pl.ds(row0, SLAB_TAIL)])

    @pl.when(s < NS - 1)
    def _write_rest():
        pltpu.sync_copy(acc.at[pl.ds(row0 + SLAB_TAIL, SLAB - SLAB_TAIL)],
                        out_hbm.at[c, pl.ds(row0 + SLAB_TAIL, SLAB - SLAB_TAIL)])


@functools.lru_cache(maxsize=1)
def _make_segsum():
  return pl.kernel(
    _segsum_body,
    out_type=jax.ShapeDtypeStruct((NC, N, D), jnp.float32),
    mesh=plsc.VectorSubcoreMesh(core_axis_name="c", subcore_axis_name="s"),
    scratch_types=[
        pltpu.VMEM((NCHUNK, CHUNK), jnp.int32),
        pltpu.VMEM((1, CHUNK), jnp.int32),
        pltpu.VMEM((1, CHUNK), jnp.int32),
        pltpu.VMEM((CHUNK, D), jnp.float32),
        pltpu.VMEM((CHUNK, D), jnp.float32),
        pltpu.VMEM_SHARED((N, D), jnp.float32),
        pltpu.SemaphoreType.DMA,
        pltpu.SemaphoreType.DMA,
        pltpu.SemaphoreType.DMA,
        pltpu.SemaphoreType.DMA,
        pltpu.SemaphoreType.DMA,
        pltpu.SemaphoreType.DMA,
    ],
    name="segsum_sc",
  )


def _tc1_body(x_ref, p0_ref, p1_ref, w1a_ref, b1a_ref, w1b_ref, b1b_ref,
              lng_ref, lnb_ref, o_ref):
    h0 = p0_ref[...] + p1_ref[...] - x_ref[...]
    t = jnp.dot(h0, w1a_ref[...], preferred_element_type=jnp.float32)
    t = jnp.maximum(t + b1a_ref[...], 0.0)
    h1 = jnp.dot(t, w1b_ref[...], preferred_element_type=jnp.float32)
    h1 = h1 + b1b_ref[...]
    m = jnp.sum(h1, axis=-1, keepdims=True) * (1.0 / D)
    d = h1 - m
    v = jnp.sum(d * d, axis=-1, keepdims=True) * (1.0 / D)
    hn = d * lax.rsqrt(v + 1e-5) * lng_ref[...] + lnb_ref[...]
    o_ref[...] = jnp.where(hn > 0, hn, 0.2 * hn)


def _tc1(x, p0, p1, w1a, b1a, w1b, b1b, lng, lnb):
    row_spec = pl.BlockSpec((BLK, D), lambda i: (i, 0))
    full = lambda a: pl.BlockSpec(a.shape, lambda i: (0,) * a.ndim)
    return pl.pallas_call(
        _tc1_body,
        grid=(NBLK,),
        in_specs=[row_spec, row_spec, row_spec,
                  full(w1a), full(b1a), full(w1b), full(b1b),
                  full(lng), full(lnb)],
        out_specs=row_spec,
        out_shape=jax.ShapeDtypeStruct((N, D), jnp.float32),
    )(x, p0, p1, w1a, b1a, w1b, b1b, lng, lnb)


def _tc2_body(h_ref, q0_ref, q1_ref, w2a_ref, b2a_ref, w2b_ref, b2b_ref,
              bt_ref, pooled_ref):
    i = pl.program_id(0)

    @pl.when(i == 0)
    def _init():
        pooled_ref[...] = jnp.full((G, D), -jnp.inf, jnp.float32)

    h0 = q0_ref[...] + q1_ref[...] - h_ref[...]
    t = jnp.dot(h0, w2a_ref[...], preferred_element_type=jnp.float32)
    t = jnp.maximum(t + b2a_ref[...], 0.0)
    h2 = jnp.dot(t, w2b_ref[...], preferred_element_type=jnp.float32)
    h2 = h2 + b2b_ref[...]
    h2 = jnp.where(h2 > 0, h2, 0.2 * h2)
    bt = bt_ref[...]
    rows = [jnp.max(jnp.where(bt == g, h2, -jnp.inf), axis=0)
            for g in range(G)]
    pooled_ref[...] = jnp.maximum(pooled_ref[...], jnp.stack(rows))


def _tc2(h, q0, q1, w2a, b2a, w2b, b2b, batch2d):
    row_spec = pl.BlockSpec((BLK, D), lambda i: (i, 0))
    full = lambda a: pl.BlockSpec(a.shape, lambda i: (0,) * a.ndim)
    return pl.pallas_call(
        _tc2_body,
        grid=(NBLK,),
        in_specs=[row_spec, row_spec, row_spec,
                  full(w2a), full(b2a), full(w2b), full(b2b),
                  pl.BlockSpec((BLK, 1), lambda i: (i, 0))],
        out_specs=pl.BlockSpec((G, D), lambda i: (0, 0)),
        out_shape=jax.ShapeDtypeStruct((G, D), jnp.float32),
    )(h, q0, q1, w2a, b2a, w2b, b2b, batch2d)


def _tc3_body(p_ref, sw1_ref, sb1_ref, slng_ref, slnb_ref, sw2_ref, sb2_ref,
              o_ref):
    z = jnp.dot(p_ref[...], sw1_ref[...], preferred_element_type=jnp.float32)
    z = z + sb1_ref[...]
    m = jnp.sum(z, axis=-1, keepdims=True) * (1.0 / H)
    d = z - m
    v = jnp.sum(d * d, axis=-1, keepdims=True) * (1.0 / H)
    z = d * lax.rsqrt(v + 1e-5) * slng_ref[...] + slnb_ref[...]
    z = jnp.where(z > 0, z, 0.2 * z)
    logits = jnp.dot(z, sw2_ref[...], preferred_element_type=jnp.float32)
    logits = logits + sb2_ref[...]
    mx = jnp.max(logits, axis=-1, keepdims=True)
    e = jnp.exp(logits - mx)
    o_ref[...] = e / jnp.sum(e, axis=-1, keepdims=True)


def _tc3(pooled, sw1, sb1, slng, slnb, sw2, sb2):
    return pl.pallas_call(
        _tc3_body,
        out_shape=jax.ShapeDtypeStruct((G, C), jnp.float32),
    )(pooled, sw1, sb1, slng, slnb, sw2, sb2)


def kernel(x, edge_index, batch, W1a, b1a, W1b, b1b, lng, lnb,
           W2a, b2a, W2b, b2b, Sw1, Sb1, Slng, Slnb, Sw2, Sb2):
    src3 = edge_index[0].reshape(NW, NCHUNK, CHUNK)
    dst3 = edge_index[1].reshape(NW * NCHUNK, CHUNK)
    _segsum = _make_segsum()
    p = _segsum(x, src3, dst3)
    h = _tc1(x, p[0], p[1], W1a, b1a.reshape(1, D), W1b, b1b.reshape(1, D),
             lng.reshape(1, D), lnb.reshape(1, D))
    q = _segsum(h, src3, dst3)
    pooled = _tc2(h, q[0], q[1], W2a, b2a.reshape(1, D), W2b,
                  b2b.reshape(1, D), batch.reshape(N, 1))
    return _tc3(pooled, Sw1, Sb1.reshape(1, H), Slng.reshape(1, H),
                Slnb.reshape(1, H), Sw2, Sb2.reshape(1, C))
